# baseline (speedup 1.0000x reference)
import functools

import jax
import numpy as np
from jax import lax as _lax


def _lane_bcast(v16, lane):
    idx = jnp.full((L, 1), lane, jnp.int32)
    dnums = _lax.GatherDimensionNumbers(
        offset_dims=(), collapsed_slice_dims=(0,), start_index_map=(0,))
    return _lax.gather(v16, idx, dimension_numbers=dnums, slice_sizes=(1,),
                       mode=_lax.GatherScatterMode.PROMISE_IN_BOUNDS)
import jax.numpy as jnp
from jax import lax
from jax.experimental import pallas as pl
from jax.experimental.pallas import tpu as pltpu
from jax.experimental.pallas import tpu_sc as plsc

NC = 2
NS = 16
NW = NC * NS
L = 16
K = 64
NBUF = 4


def _i32(x):
    return x.astype(jnp.int32)


BC = 16


def _make_sc_agg(n_pad, emb, nblk0, nblk1, compute_deg):
    mesh = plsc.VectorSubcoreMesh(core_axis_name="c", subcore_axis_name="s",
                                  num_cores=NC, num_subcores=NS)
    stripe = n_pad // NS

    out_type = [jax.ShapeDtypeStruct((NC, n_pad, emb), jnp.float32)]
    if compute_deg:
        out_type.append(jax.ShapeDtypeStruct((NC, n_pad), jnp.float32))

    scratch = dict(
        src_v=pltpu.VMEM((BC, K), jnp.int32),
        dst_v=pltpu.VMEM((BC, K), jnp.int32),
        w_v=pltpu.VMEM((BC, K), jnp.float32),
        agg_sh=pltpu.VMEM_SHARED((n_pad, emb), jnp.float32),
        sd=pltpu.SemaphoreType.DMA,
        **{f"rows{r}": pltpu.VMEM((K, emb // 2), jnp.int32)
           for r in range(NBUF)},
        **{f"sg{r}": pltpu.SemaphoreType.DMA for r in range(NBUF)},
        **{f"fb{r}": pltpu.VMEM((K, emb), jnp.float32) for r in range(2)},
        **{f"ss{r}": pltpu.SemaphoreType.DMA for r in range(2)},
    )
    if compute_deg:
        scratch.update(
            zbuf_v=pltpu.VMEM((stripe,), jnp.float32),
            deg_sh=pltpu.VMEM_SHARED((n_pad,), jnp.float32),
        )

    def body(x_hbm, src_hbm, dst_hbm, w_hbm, *outs, src_v, dst_v, w_v,
             agg_sh, sd, zbuf_v=None, deg_sh=None, **bufs):
        if compute_deg:
            agg_out, deg_out = outs
        else:
            (agg_out,) = outs
        c = lax.axis_index("c")
        s = lax.axis_index("s")
        nblk_c = jnp.where(c == 0, nblk0, nblk1)
        bbase = jnp.where(c == 0, 0, nblk0)
        rows = [bufs[f"rows{r}"] for r in range(NBUF)]
        sg = [bufs[f"sg{r}"] for r in range(NBUF)]
        fbuf = [bufs[f"fb{r}"] for r in range(2)]
        ss = [bufs[f"ss{r}"] for r in range(2)]

        def zrow(e, _):
            for j in range(emb // L):
                fbuf[0][e, pl.ds(j * L, L)] = jnp.zeros((L,), jnp.float32)
            return 0
        lax.fori_loop(0, K, zrow, 0)
        for q in range(stripe // K):
            off = s * stripe + q * K
            pltpu.sync_copy(fbuf[0], agg_sh.at[pl.ds(off, K)])
        if compute_deg:
            for i in range(stripe // L):
                zbuf_v[pl.ds(i * L, L)] = jnp.zeros((L,), jnp.float32)
            pltpu.sync_copy(zbuf_v, deg_sh.at[pl.ds(s * stripe, stripe)])
        plsc.subcore_barrier()

        def _scale_rows(buf, fb, j):
            def scale(g, _):
                w16 = w_v[j, pl.ds(g * L, L)]
                for l in range(L):
                    e = g * L + l
                    wb = _lane_bcast(w16, l)
                    for h in range(emb // (2 * L)):
                        w32 = rows[buf][e, pl.ds(h * L, L)]
                        ev = plsc.bitcast(w32 << 16, jnp.float32)
                        od = plsc.bitcast(
                            w32 & jnp.int32(-65536), jnp.float32)
                        fbuf[fb][e, pl.ds(h * 2 * L, L)] = ev * wb
                        fbuf[fb][e, pl.ds(h * 2 * L + L, L)] = od * wb
                return 0
            lax.fori_loop(0, K // L, scale, 0)

        def _drain_scatters():
            for par in range(2):
                pltpu.make_async_copy(
                    fbuf[par], agg_sh.at[dst_v.at[0]], ss[par]).wait()

        def _drain_deg():
            for _j in range(BC):
                pltpu.make_async_copy(
                    w_v.at[0], deg_sh.at[dst_v.at[0]], sd).wait()

        def block(b, _):
            @pl.when(b > 0)
            def _():
                _drain_scatters()
                if compute_deg:
                    _drain_deg()

            boff = pl.multiple_of((bbase + b) * BC, BC)
            pltpu.sync_copy(src_hbm.at[s, pl.ds(boff, BC)], src_v)
            pltpu.sync_copy(dst_hbm.at[s, pl.ds(boff, BC)], dst_v)
            pltpu.sync_copy(w_hbm.at[s, pl.ds(boff, BC)], w_v)

            gd = [None] * NBUF
            for j0 in range(NBUF - 1):
                gd[j0] = pltpu.async_copy(
                    x_hbm.at[src_v.at[j0]], rows[j0], sg[j0])
            for j in range(BC):
                buf = j % NBUF
                fb = j % 2
                if j + NBUF - 1 < BC:
                    nb = (j + NBUF - 1) % NBUF
                    gd[nb] = pltpu.async_copy(
                        x_hbm.at[src_v.at[j + NBUF - 1]], rows[nb], sg[nb])
                gd[buf].wait()
                if j >= 2:
                    pltpu.make_async_copy(
                        fbuf[fb], agg_sh.at[dst_v.at[0]], ss[fb]).wait()
                _scale_rows(buf, fb, j)
                pltpu.async_copy(fbuf[fb], agg_sh.at[dst_v.at[j]], ss[fb],
                                 add=True)
                if compute_deg:
                    pltpu.async_copy(w_v.at[j], deg_sh.at[dst_v.at[j]], sd,
                                     add=True)
            return 0
        lax.fori_loop(0, nblk_c, block, 0)
        _drain_scatters()
        if compute_deg:
            _drain_deg()
        plsc.subcore_barrier()

        off = s * stripe
        pltpu.sync_copy(agg_sh.at[pl.ds(off, stripe)],
                        agg_out.at[c, pl.ds(off, stripe)])
        if compute_deg:
            pltpu.sync_copy(deg_sh.at[pl.ds(off, stripe)],
                            deg_out.at[c, pl.ds(off, stripe)])

    return pl.kernel(
        body, out_type=tuple(out_type), mesh=mesh, scratch_types=scratch,
        compiler_params=pltpu.CompilerParams(needs_layout_passes=False,
                                             use_tc_tiling_on_sc=False))


def _tc_layer(x, aggp, degp3, w_root, w_neigh, b, blk=1024):
    n_pad, emb = x.shape
    grid = n_pad // blk

    def body(x_ref, a_ref, d_ref, wr_ref, wn_ref, b_ref, o_ref):
        agg = a_ref[0] + a_ref[1]
        deg = d_ref[0] + d_ref[1]
        inv = 1.0 / jnp.maximum(deg, 1e-12)
        h = (jnp.dot(x_ref[...], wr_ref[...],
                     preferred_element_type=jnp.float32,
                     precision=lax.Precision.HIGHEST)
             + jnp.dot(agg * inv, wn_ref[...],
                       preferred_element_type=jnp.float32,
                       precision=lax.Precision.HIGHEST)
             + b_ref[...])
        o_ref[...] = jnp.maximum(h, 0.0)

    return pl.pallas_call(
        body,
        grid=(grid,),
        in_specs=[
            pl.BlockSpec((blk, emb), lambda i: (i, 0)),
            pl.BlockSpec((NC, blk, emb), lambda i: (0, i, 0)),
            pl.BlockSpec((NC, blk, 1), lambda i: (0, i, 0)),
            pl.BlockSpec((emb, emb), lambda i: (0, 0)),
            pl.BlockSpec((emb, emb), lambda i: (0, 0)),
            pl.BlockSpec((1, emb), lambda i: (0, 0)),
        ],
        out_specs=pl.BlockSpec((blk, emb), lambda i: (i, 0)),
        out_shape=jax.ShapeDtypeStruct((n_pad, emb), jnp.float32),
    )(x, aggp, degp3, w_root, w_neigh, b)


def _make_sc_pairprod(emb, bq):
    mesh = plsc.VectorSubcoreMesh(core_axis_name="c", subcore_axis_name="s",
                                  num_cores=NC, num_subcores=NS)
    qpt = bq // NW

    scratch = dict(
        ia_v=pltpu.VMEM((qpt,), jnp.int32),
        ib_v=pltpu.VMEM((qpt,), jnp.int32),
        ie_v=pltpu.VMEM((qpt,), jnp.int32),
        ha_v=pltpu.VMEM((qpt, emb), jnp.float32),
        hb_v=pltpu.VMEM((qpt, emb), jnp.float32),
        ef_v=pltpu.VMEM((qpt, emb), jnp.float32),
        sem=pltpu.SemaphoreType.DMA,
    )

    def body(h_hbm, eff_hbm, ia_hbm, ib_hbm, ie_hbm, p_hbm,
             *, ia_v, ib_v, ie_v, ha_v, hb_v, ef_v, sem):
        c = lax.axis_index("c")
        s = lax.axis_index("s")
        wid = c * NS + s

        pltpu.sync_copy(ia_hbm.at[wid], ia_v)
        pltpu.sync_copy(ib_hbm.at[wid], ib_v)
        pltpu.sync_copy(ie_hbm.at[wid], ie_v)

        ca = pltpu.async_copy(h_hbm.at[ia_v], ha_v, sem)
        cb = pltpu.async_copy(h_hbm.at[ib_v], hb_v, sem)
        ce = pltpu.async_copy(eff_hbm.at[ie_v], ef_v, sem)
        ca.wait()
        cb.wait()
        ce.wait()

        def qstep(q, _):
            for j in range(emb // L):
                sl = pl.ds(j * L, L)
                a = ha_v[q, sl]
                bb = hb_v[q, sl]
                ee = ef_v[q, sl]
                ha_v[q, sl] = (a + ee) * (bb + ee)
            return 0
        lax.fori_loop(0, qpt, qstep, 0)

        pltpu.sync_copy(ha_v, p_hbm.at[pl.ds(wid * qpt, qpt)])

    return pl.kernel(body,
                     out_type=jax.ShapeDtypeStruct((bq, emb), jnp.float32),
                     mesh=mesh, scratch_types=scratch)


def _tc_decode(p, dec_w, dec_b2):
    bq, emb = p.shape

    def body(p_ref, w_ref, b_ref, o_ref):
        z = jnp.dot(p_ref[...], w_ref[...],
                    preferred_element_type=jnp.float32,
                    precision=lax.Precision.HIGHEST) + b_ref[...]
        o_ref[...] = 1.0 / (1.0 + jnp.exp(-z))

    return pl.pallas_call(
        body,
        out_shape=jax.ShapeDtypeStruct((bq, 1), jnp.float32),
    )(p, dec_w, dec_b2)


def kernel(graph_x, edge_index, edge_weight, x_nodes, effect_ids, effect_table,
           W_root0, W_neigh0, b0, W_root1, W_neigh1, b1, dec_W, dec_b):
    n, emb = graph_x.shape
    e = edge_weight.shape[0]
    bq = x_nodes.shape[0]

    n_pad = ((n + NS * K - 1) // (NS * K)) * (NS * K)
    epw = NS * K * BC
    nblk_t = (e + epw - 1) // epw
    e_pad = nblk_t * epw
    chunks = e_pad // (NS * K)
    nblk0 = max(1, min(nblk_t - 1, round(nblk_t * 0.70)))
    nblk1 = nblk_t - nblk0

    x_p = jnp.concatenate(
        [graph_x, jnp.zeros((n_pad - n, emb), jnp.float32)], axis=0)

    src = _i32(edge_index[0])
    dst = _i32(edge_index[1])
    pad = e_pad - e
    if pad:
        src = jnp.concatenate([src, jnp.zeros((pad,), jnp.int32)])
        dst = jnp.concatenate([dst, jnp.zeros((pad,), jnp.int32)])
        w = jnp.concatenate([edge_weight, jnp.zeros((pad,), jnp.float32)])
    else:
        w = edge_weight
    src_r = src.reshape(NS, chunks, K)
    dst_r = dst.reshape(NS, chunks, K)
    w_r = w.reshape(NS, chunks, K)

    sc_agg_a = _make_sc_agg(n_pad, emb, nblk0, nblk1, compute_deg=True)
    sc_agg_c = _make_sc_agg(n_pad, emb, nblk0, nblk1, compute_deg=False)

    perm = np.arange(emb).reshape(emb // (2 * L), L, 2).transpose(0, 2, 1)
    perm = perm.reshape(emb)
    wn0 = W_neigh0[perm, :]
    wn1 = W_neigh1[perm, :]

    def _pack_bf16(arr):
        bf = arr.astype(jnp.bfloat16).reshape(n_pad, emb // 2, 2)
        return jax.lax.bitcast_convert_type(bf, jnp.int32)

    aggp0, degp = sc_agg_a(_pack_bf16(x_p), src_r, dst_r, w_r)
    degp3 = degp.reshape(NC, n_pad, 1)

    b0r = b0.reshape(1, emb)
    b1r = b1.reshape(1, emb)
    h1 = _tc_layer(x_p, aggp0, degp3, W_root0, wn0, b0r)
    (aggp1,) = sc_agg_c(_pack_bf16(h1), src_r, dst_r, w_r)
    h2 = _tc_layer(h1, aggp1, degp3, W_root1, wn1, b1r)

    qpt = bq // NW
    ia = _i32(x_nodes[:, 0]).reshape(NW, qpt)
    ib = _i32(x_nodes[:, 1]).reshape(NW, qpt)
    ie = _i32(effect_ids).reshape(NW, qpt)

    sc_pp = _make_sc_pairprod(emb, bq)
    p = sc_pp(h2, effect_table, ia, ib, ie)
    return _tc_decode(p, dec_W, dec_b.reshape(1, 1))


# speedup vs baseline: 1.0966x; 1.0966x over previous
import functools

import jax
import numpy as np
from jax import lax as _lax


def _lane_bcast(v16, lane):
    idx = jnp.full((L, 1), lane, jnp.int32)
    dnums = _lax.GatherDimensionNumbers(
        offset_dims=(), collapsed_slice_dims=(0,), start_index_map=(0,))
    return _lax.gather(v16, idx, dimension_numbers=dnums, slice_sizes=(1,),
                       mode=_lax.GatherScatterMode.PROMISE_IN_BOUNDS)
import jax.numpy as jnp
from jax import lax
from jax.experimental import pallas as pl
from jax.experimental.pallas import tpu as pltpu
from jax.experimental.pallas import tpu_sc as plsc

NC = 2
NS = 16
NW = NC * NS
L = 16
K = 64
NBUF = 4


def _i32(x):
    return x.astype(jnp.int32)


BC = 16


def _make_sc_agg(n_pad, emb, nblk0, nblk1, compute_deg):
    mesh = plsc.VectorSubcoreMesh(core_axis_name="c", subcore_axis_name="s",
                                  num_cores=NC, num_subcores=NS)
    stripe = n_pad // NS

    out_type = [jax.ShapeDtypeStruct((NC, n_pad, emb), jnp.float32)]
    if compute_deg:
        out_type.append(jax.ShapeDtypeStruct((NC, n_pad), jnp.float32))

    scratch = dict(
        src_v=pltpu.VMEM((BC, K), jnp.int32),
        dst_v=pltpu.VMEM((BC, K), jnp.int32),
        w_v=pltpu.VMEM((BC, K), jnp.float32),
        agg_sh=pltpu.VMEM_SHARED((n_pad, emb), jnp.float32),
        sd=pltpu.SemaphoreType.DMA,
        **{f"rows{r}": pltpu.VMEM((K, emb), jnp.float32) for r in range(NBUF)},
        **{f"sg{r}": pltpu.SemaphoreType.DMA for r in range(NBUF)},
        **{f"ss{r}": pltpu.SemaphoreType.DMA for r in range(NBUF)},
    )
    if compute_deg:
        scratch.update(
            zbuf_v=pltpu.VMEM((stripe,), jnp.float32),
            deg_sh=pltpu.VMEM_SHARED((n_pad,), jnp.float32),
        )

    def body(x_hbm, src_hbm, dst_hbm, w_hbm, *outs, src_v, dst_v, w_v,
             agg_sh, sd, zbuf_v=None, deg_sh=None, **bufs):
        if compute_deg:
            agg_out, deg_out = outs
        else:
            (agg_out,) = outs
        c = lax.axis_index("c")
        s = lax.axis_index("s")
        nblk_c = jnp.where(c == 0, nblk0, nblk1)
        bbase = jnp.where(c == 0, 0, nblk0)
        rows = [bufs[f"rows{r}"] for r in range(NBUF)]
        sg = [bufs[f"sg{r}"] for r in range(NBUF)]
        ss = [bufs[f"ss{r}"] for r in range(NBUF)]

        def zrow(e, _):
            for j in range(emb // L):
                rows[0][e, pl.ds(j * L, L)] = jnp.zeros((L,), jnp.float32)
            return 0
        lax.fori_loop(0, K, zrow, 0)
        for q in range(stripe // K):
            off = s * stripe + q * K
            pltpu.sync_copy(rows[0], agg_sh.at[pl.ds(off, K)])
        if compute_deg:
            for i in range(stripe // L):
                zbuf_v[pl.ds(i * L, L)] = jnp.zeros((L,), jnp.float32)
            pltpu.sync_copy(zbuf_v, deg_sh.at[pl.ds(s * stripe, stripe)])
        plsc.subcore_barrier()

        def _scale_rows(buf, j):
            def scale(g, _):
                w16 = w_v[j, pl.ds(g * L, L)]
                for l in range(L):
                    e = g * L + l
                    wb = _lane_bcast(w16, l)
                    for jj in range(emb // L):
                        sl = pl.ds(jj * L, L)
                        rows[buf][e, sl] = rows[buf][e, sl] * wb
                return 0
            lax.fori_loop(0, K // L, scale, 0)

        def _drain_scatters():
            for par in range(NBUF):
                pltpu.make_async_copy(
                    rows[par], agg_sh.at[dst_v.at[0]], ss[par]).wait()

        def _drain_deg():
            for _j in range(BC):
                pltpu.make_async_copy(
                    w_v.at[0], deg_sh.at[dst_v.at[0]], sd).wait()

        def block(b, _):
            @pl.when(b > 0)
            def _():
                _drain_scatters()
                if compute_deg:
                    _drain_deg()

            boff = pl.multiple_of((bbase + b) * BC, BC)
            pltpu.sync_copy(src_hbm.at[s, pl.ds(boff, BC)], src_v)
            pltpu.sync_copy(dst_hbm.at[s, pl.ds(boff, BC)], dst_v)
            pltpu.sync_copy(w_hbm.at[s, pl.ds(boff, BC)], w_v)

            gd = [None] * NBUF
            for j0 in range(NBUF - 1):
                gd[j0] = pltpu.async_copy(
                    x_hbm.at[src_v.at[j0]], rows[j0], sg[j0])
            for j in range(BC):
                buf = j % NBUF
                if j + NBUF - 1 < BC:
                    nb = (j + NBUF - 1) % NBUF
                    if j > 0:
                        pltpu.make_async_copy(
                            rows[nb], agg_sh.at[dst_v.at[0]], ss[nb]).wait()
                    gd[nb] = pltpu.async_copy(
                        x_hbm.at[src_v.at[j + NBUF - 1]], rows[nb], sg[nb])
                gd[buf].wait()
                _scale_rows(buf, j)
                pltpu.async_copy(rows[buf], agg_sh.at[dst_v.at[j]], ss[buf],
                                 add=True)
                if compute_deg:
                    pltpu.async_copy(w_v.at[j], deg_sh.at[dst_v.at[j]], sd,
                                     add=True)
            return 0
        lax.fori_loop(0, nblk_c, block, 0)
        _drain_scatters()
        if compute_deg:
            _drain_deg()
        plsc.subcore_barrier()

        off = s * stripe
        pltpu.sync_copy(agg_sh.at[pl.ds(off, stripe)],
                        agg_out.at[c, pl.ds(off, stripe)])
        if compute_deg:
            pltpu.sync_copy(deg_sh.at[pl.ds(off, stripe)],
                            deg_out.at[c, pl.ds(off, stripe)])

    return pl.kernel(
        body, out_type=tuple(out_type), mesh=mesh, scratch_types=scratch,
        compiler_params=pltpu.CompilerParams(needs_layout_passes=False,
                                             use_tc_tiling_on_sc=False))


def _tc_layer(x, aggp, degp3, w_root, w_neigh, b, blk=1024):
    n_pad, emb = x.shape
    grid = n_pad // blk

    def body(x_ref, a_ref, d_ref, wr_ref, wn_ref, b_ref, o_ref):
        agg = a_ref[0] + a_ref[1]
        deg = d_ref[0] + d_ref[1]
        inv = 1.0 / jnp.maximum(deg, 1e-12)
        h = (jnp.dot(x_ref[...], wr_ref[...],
                     preferred_element_type=jnp.float32,
                     precision=lax.Precision.HIGHEST)
             + jnp.dot(agg * inv, wn_ref[...],
                       preferred_element_type=jnp.float32,
                       precision=lax.Precision.HIGHEST)
             + b_ref[...])
        o_ref[...] = jnp.maximum(h, 0.0)

    return pl.pallas_call(
        body,
        grid=(grid,),
        in_specs=[
            pl.BlockSpec((blk, emb), lambda i: (i, 0)),
            pl.BlockSpec((NC, blk, emb), lambda i: (0, i, 0)),
            pl.BlockSpec((NC, blk, 1), lambda i: (0, i, 0)),
            pl.BlockSpec((emb, emb), lambda i: (0, 0)),
            pl.BlockSpec((emb, emb), lambda i: (0, 0)),
            pl.BlockSpec((1, emb), lambda i: (0, 0)),
        ],
        out_specs=pl.BlockSpec((blk, emb), lambda i: (i, 0)),
        out_shape=jax.ShapeDtypeStruct((n_pad, emb), jnp.float32),
    )(x, aggp, degp3, w_root, w_neigh, b)


def _make_sc_pairprod(emb, bq):
    mesh = plsc.VectorSubcoreMesh(core_axis_name="c", subcore_axis_name="s",
                                  num_cores=NC, num_subcores=NS)
    qpt = bq // NW

    scratch = dict(
        ia_v=pltpu.VMEM((qpt,), jnp.int32),
        ib_v=pltpu.VMEM((qpt,), jnp.int32),
        ie_v=pltpu.VMEM((qpt,), jnp.int32),
        ha_v=pltpu.VMEM((qpt, emb), jnp.float32),
        hb_v=pltpu.VMEM((qpt, emb), jnp.float32),
        ef_v=pltpu.VMEM((qpt, emb), jnp.float32),
        sem=pltpu.SemaphoreType.DMA,
    )

    def body(h_hbm, eff_hbm, ia_hbm, ib_hbm, ie_hbm, p_hbm,
             *, ia_v, ib_v, ie_v, ha_v, hb_v, ef_v, sem):
        c = lax.axis_index("c")
        s = lax.axis_index("s")
        wid = c * NS + s

        pltpu.sync_copy(ia_hbm.at[wid], ia_v)
        pltpu.sync_copy(ib_hbm.at[wid], ib_v)
        pltpu.sync_copy(ie_hbm.at[wid], ie_v)

        ca = pltpu.async_copy(h_hbm.at[ia_v], ha_v, sem)
        cb = pltpu.async_copy(h_hbm.at[ib_v], hb_v, sem)
        ce = pltpu.async_copy(eff_hbm.at[ie_v], ef_v, sem)
        ca.wait()
        cb.wait()
        ce.wait()

        def qstep(q, _):
            for j in range(emb // L):
                sl = pl.ds(j * L, L)
                a = ha_v[q, sl]
                bb = hb_v[q, sl]
                ee = ef_v[q, sl]
                ha_v[q, sl] = (a + ee) * (bb + ee)
            return 0
        lax.fori_loop(0, qpt, qstep, 0)

        pltpu.sync_copy(ha_v, p_hbm.at[pl.ds(wid * qpt, qpt)])

    return pl.kernel(body,
                     out_type=jax.ShapeDtypeStruct((bq, emb), jnp.float32),
                     mesh=mesh, scratch_types=scratch)


def _tc_decode(p, dec_w, dec_b2):
    bq, emb = p.shape

    def body(p_ref, w_ref, b_ref, o_ref):
        z = jnp.dot(p_ref[...], w_ref[...],
                    preferred_element_type=jnp.float32,
                    precision=lax.Precision.HIGHEST) + b_ref[...]
        o_ref[...] = 1.0 / (1.0 + jnp.exp(-z))

    return pl.pallas_call(
        body,
        out_shape=jax.ShapeDtypeStruct((bq, 1), jnp.float32),
    )(p, dec_w, dec_b2)


def kernel(graph_x, edge_index, edge_weight, x_nodes, effect_ids, effect_table,
           W_root0, W_neigh0, b0, W_root1, W_neigh1, b1, dec_W, dec_b):
    n, emb = graph_x.shape
    e = edge_weight.shape[0]
    bq = x_nodes.shape[0]

    n_pad = ((n + NS * K - 1) // (NS * K)) * (NS * K)
    epw = NS * K * BC
    nblk_t = (e + epw - 1) // epw
    e_pad = nblk_t * epw
    chunks = e_pad // (NS * K)
    nblk0 = max(1, min(nblk_t - 1, round(nblk_t * 0.70)))
    nblk1 = nblk_t - nblk0

    x_p = jnp.concatenate(
        [graph_x, jnp.zeros((n_pad - n, emb), jnp.float32)], axis=0)

    src = _i32(edge_index[0])
    dst = _i32(edge_index[1])
    pad = e_pad - e
    if pad:
        src = jnp.concatenate([src, jnp.zeros((pad,), jnp.int32)])
        dst = jnp.concatenate([dst, jnp.zeros((pad,), jnp.int32)])
        w = jnp.concatenate([edge_weight, jnp.zeros((pad,), jnp.float32)])
    else:
        w = edge_weight
    src_r = src.reshape(NS, chunks, K)
    dst_r = dst.reshape(NS, chunks, K)
    w_r = w.reshape(NS, chunks, K)

    sc_agg_a = _make_sc_agg(n_pad, emb, nblk0, nblk1, compute_deg=True)
    sc_agg_c = _make_sc_agg(n_pad, emb, nblk0, nblk1, compute_deg=False)

    aggp0, degp = sc_agg_a(x_p, src_r, dst_r, w_r)
    degp3 = degp.reshape(NC, n_pad, 1)

    b0r = b0.reshape(1, emb)
    b1r = b1.reshape(1, emb)
    h1 = _tc_layer(x_p, aggp0, degp3, W_root0, W_neigh0, b0r)
    (aggp1,) = sc_agg_c(h1, src_r, dst_r, w_r)
    h2 = _tc_layer(h1, aggp1, degp3, W_root1, W_neigh1, b1r)

    qpt = bq // NW
    ia = _i32(x_nodes[:, 0]).reshape(NW, qpt)
    ib = _i32(x_nodes[:, 1]).reshape(NW, qpt)
    ie = _i32(effect_ids).reshape(NW, qpt)

    sc_pp = _make_sc_pairprod(emb, bq)
    p = sc_pp(h2, effect_table, ia, ib, ie)
    return _tc_decode(p, dec_W, dec_b.reshape(1, 1))


# speedup vs baseline: 1.1428x; 1.0422x over previous
import functools

import jax
import numpy as np
from jax import lax as _lax


def _lane_bcast(v16, lane):
    idx = jnp.full((L, 1), lane, jnp.int32)
    dnums = _lax.GatherDimensionNumbers(
        offset_dims=(), collapsed_slice_dims=(0,), start_index_map=(0,))
    return _lax.gather(v16, idx, dimension_numbers=dnums, slice_sizes=(1,),
                       mode=_lax.GatherScatterMode.PROMISE_IN_BOUNDS)
import jax.numpy as jnp
from jax import lax
from jax.experimental import pallas as pl
from jax.experimental.pallas import tpu as pltpu
from jax.experimental.pallas import tpu_sc as plsc

NC = 2
NS = 16
NW = NC * NS
L = 16
K = 64
NBUF = 4


def _i32(x):
    return x.astype(jnp.int32)


BC = 16


def _make_sc_agg(n_pad, emb, nblk0, nblk1, compute_deg):
    mesh = plsc.VectorSubcoreMesh(core_axis_name="c", subcore_axis_name="s",
                                  num_cores=NC, num_subcores=NS)
    stripe = n_pad // NS

    out_type = [jax.ShapeDtypeStruct((NC, n_pad, emb), jnp.float32)]
    if compute_deg:
        out_type.append(jax.ShapeDtypeStruct((NC, n_pad), jnp.float32))

    scratch = dict(
        src_v=pltpu.VMEM((BC, K), jnp.int32),
        dst_v=pltpu.VMEM((BC, K), jnp.int32),
        w_v=pltpu.VMEM((BC, K), jnp.float32),
        agg_sh=pltpu.VMEM_SHARED((n_pad, emb), jnp.float32),
        sd=pltpu.SemaphoreType.DMA,
        **{f"rows{r}": pltpu.VMEM((K, emb), jnp.float32) for r in range(NBUF)},
        **{f"sg{r}": pltpu.SemaphoreType.DMA for r in range(NBUF)},
        **{f"ss{r}": pltpu.SemaphoreType.DMA for r in range(NBUF)},
    )
    if compute_deg:
        scratch.update(
            zbuf_v=pltpu.VMEM((stripe,), jnp.float32),
            deg_sh=pltpu.VMEM_SHARED((n_pad,), jnp.float32),
        )

    def body(x_hbm, src_hbm, dst_hbm, w_hbm, *outs, src_v, dst_v, w_v,
             agg_sh, sd, zbuf_v=None, deg_sh=None, **bufs):
        if compute_deg:
            agg_out, deg_out = outs
        else:
            (agg_out,) = outs
        c = lax.axis_index("c")
        s = lax.axis_index("s")
        nblk_c = jnp.where(c == 0, nblk0, nblk1)
        bbase = jnp.where(c == 0, 0, nblk0)
        rows = [bufs[f"rows{r}"] for r in range(NBUF)]
        sg = [bufs[f"sg{r}"] for r in range(NBUF)]
        ss = [bufs[f"ss{r}"] for r in range(NBUF)]

        def zrow(e, _):
            for j in range(emb // L):
                rows[0][e, pl.ds(j * L, L)] = jnp.zeros((L,), jnp.float32)
            return 0
        lax.fori_loop(0, K, zrow, 0)
        for q in range(stripe // K):
            off = s * stripe + q * K
            pltpu.sync_copy(rows[0], agg_sh.at[pl.ds(off, K)])
        if compute_deg:
            for i in range(stripe // L):
                zbuf_v[pl.ds(i * L, L)] = jnp.zeros((L,), jnp.float32)
            pltpu.sync_copy(zbuf_v, deg_sh.at[pl.ds(s * stripe, stripe)])
        plsc.subcore_barrier()

        def _scale_rows(buf, j):
            def scale(g, _):
                w16 = w_v[j, pl.ds(g * L, L)]
                for l in range(L):
                    e = g * L + l
                    wb = _lane_bcast(w16, l)
                    for jj in range(emb // L):
                        sl = pl.ds(jj * L, L)
                        rows[buf][e, sl] = rows[buf][e, sl] * wb
                return 0
            lax.fori_loop(0, K // L, scale, 0)

        def _drain_scatters():
            for par in range(NBUF):
                pltpu.make_async_copy(
                    rows[par], agg_sh.at[dst_v.at[0]], ss[par]).wait()

        def _drain_deg():
            for _j in range(BC):
                pltpu.make_async_copy(
                    w_v.at[0], deg_sh.at[dst_v.at[0]], sd).wait()

        def block(b, _):
            @pl.when(b > 0)
            def _():
                _drain_scatters()
                if compute_deg:
                    _drain_deg()

            boff = pl.multiple_of((bbase + b) * BC, BC)
            pltpu.sync_copy(src_hbm.at[s, pl.ds(boff, BC)], src_v)
            pltpu.sync_copy(dst_hbm.at[s, pl.ds(boff, BC)], dst_v)
            pltpu.sync_copy(w_hbm.at[s, pl.ds(boff, BC)], w_v)

            gd = [None] * NBUF
            for j0 in range(NBUF - 1):
                gd[j0] = pltpu.async_copy(
                    x_hbm.at[src_v.at[j0]], rows[j0], sg[j0])
            for j in range(BC):
                buf = j % NBUF
                if j + NBUF - 1 < BC:
                    nb = (j + NBUF - 1) % NBUF
                    if j > 0:
                        pltpu.make_async_copy(
                            rows[nb], agg_sh.at[dst_v.at[0]], ss[nb]).wait()
                    gd[nb] = pltpu.async_copy(
                        x_hbm.at[src_v.at[j + NBUF - 1]], rows[nb], sg[nb])
                gd[buf].wait()
                _scale_rows(buf, j)
                pltpu.async_copy(rows[buf], agg_sh.at[dst_v.at[j]], ss[buf],
                                 add=True)
                if compute_deg:
                    pltpu.async_copy(w_v.at[j], deg_sh.at[dst_v.at[j]], sd,
                                     add=True)
            return 0
        lax.fori_loop(0, nblk_c, block, 0)
        _drain_scatters()
        if compute_deg:
            _drain_deg()
        plsc.subcore_barrier()

        off = s * stripe
        pltpu.sync_copy(agg_sh.at[pl.ds(off, stripe)],
                        agg_out.at[c, pl.ds(off, stripe)])
        if compute_deg:
            pltpu.sync_copy(deg_sh.at[pl.ds(off, stripe)],
                            deg_out.at[c, pl.ds(off, stripe)])

    return pl.kernel(body, out_type=tuple(out_type), mesh=mesh,
                     scratch_types=scratch)


def _tc_layer(x, aggp, degp3, w_root, w_neigh, b, blk=1024):
    n_pad, emb = x.shape
    grid = n_pad // blk

    def body(x_ref, a_ref, d_ref, wr_ref, wn_ref, b_ref, o_ref):
        agg = a_ref[0] + a_ref[1]
        deg = d_ref[0] + d_ref[1]
        inv = 1.0 / jnp.maximum(deg, 1e-12)
        h = (jnp.dot(x_ref[...], wr_ref[...],
                     preferred_element_type=jnp.float32,
                     precision=lax.Precision.HIGHEST)
             + jnp.dot(agg * inv, wn_ref[...],
                       preferred_element_type=jnp.float32,
                       precision=lax.Precision.HIGHEST)
             + b_ref[...])
        o_ref[...] = jnp.maximum(h, 0.0)

    return pl.pallas_call(
        body,
        grid=(grid,),
        in_specs=[
            pl.BlockSpec((blk, emb), lambda i: (i, 0)),
            pl.BlockSpec((NC, blk, emb), lambda i: (0, i, 0)),
            pl.BlockSpec((NC, blk, 1), lambda i: (0, i, 0)),
            pl.BlockSpec((emb, emb), lambda i: (0, 0)),
            pl.BlockSpec((emb, emb), lambda i: (0, 0)),
            pl.BlockSpec((1, emb), lambda i: (0, 0)),
        ],
        out_specs=pl.BlockSpec((blk, emb), lambda i: (i, 0)),
        out_shape=jax.ShapeDtypeStruct((n_pad, emb), jnp.float32),
    )(x, aggp, degp3, w_root, w_neigh, b)


def _make_sc_pairprod(emb, bq):
    mesh = plsc.VectorSubcoreMesh(core_axis_name="c", subcore_axis_name="s",
                                  num_cores=NC, num_subcores=NS)
    qpt = bq // NW

    scratch = dict(
        ia_v=pltpu.VMEM((qpt,), jnp.int32),
        ib_v=pltpu.VMEM((qpt,), jnp.int32),
        ie_v=pltpu.VMEM((qpt,), jnp.int32),
        ha_v=pltpu.VMEM((qpt, emb), jnp.float32),
        hb_v=pltpu.VMEM((qpt, emb), jnp.float32),
        ef_v=pltpu.VMEM((qpt, emb), jnp.float32),
        sem=pltpu.SemaphoreType.DMA,
    )

    def body(h_hbm, eff_hbm, ia_hbm, ib_hbm, ie_hbm, p_hbm,
             *, ia_v, ib_v, ie_v, ha_v, hb_v, ef_v, sem):
        c = lax.axis_index("c")
        s = lax.axis_index("s")
        wid = c * NS + s

        pltpu.sync_copy(ia_hbm.at[wid], ia_v)
        pltpu.sync_copy(ib_hbm.at[wid], ib_v)
        pltpu.sync_copy(ie_hbm.at[wid], ie_v)

        ca = pltpu.async_copy(h_hbm.at[ia_v], ha_v, sem)
        cb = pltpu.async_copy(h_hbm.at[ib_v], hb_v, sem)
        ce = pltpu.async_copy(eff_hbm.at[ie_v], ef_v, sem)
        ca.wait()
        cb.wait()
        ce.wait()

        def qstep(q, _):
            for j in range(emb // L):
                sl = pl.ds(j * L, L)
                a = ha_v[q, sl]
                bb = hb_v[q, sl]
                ee = ef_v[q, sl]
                ha_v[q, sl] = (a + ee) * (bb + ee)
            return 0
        lax.fori_loop(0, qpt, qstep, 0)

        pltpu.sync_copy(ha_v, p_hbm.at[pl.ds(wid * qpt, qpt)])

    return pl.kernel(body,
                     out_type=jax.ShapeDtypeStruct((bq, emb), jnp.float32),
                     mesh=mesh, scratch_types=scratch)


def _tc_decode(p, dec_w, dec_b2):
    bq, emb = p.shape

    def body(p_ref, w_ref, b_ref, o_ref):
        z = jnp.dot(p_ref[...], w_ref[...],
                    preferred_element_type=jnp.float32,
                    precision=lax.Precision.HIGHEST) + b_ref[...]
        o_ref[...] = 1.0 / (1.0 + jnp.exp(-z))

    return pl.pallas_call(
        body,
        out_shape=jax.ShapeDtypeStruct((bq, 1), jnp.float32),
    )(p, dec_w, dec_b2)


def kernel(graph_x, edge_index, edge_weight, x_nodes, effect_ids, effect_table,
           W_root0, W_neigh0, b0, W_root1, W_neigh1, b1, dec_W, dec_b):
    n, emb = graph_x.shape
    e = edge_weight.shape[0]
    bq = x_nodes.shape[0]

    n_pad = ((n + NS * K - 1) // (NS * K)) * (NS * K)
    epw = NS * K * BC
    nblk_t = (e + epw - 1) // epw
    e_pad = nblk_t * epw
    chunks = e_pad // (NS * K)
    nblk0 = max(1, min(nblk_t - 1, round(nblk_t * 0.70)))
    nblk1 = nblk_t - nblk0

    x_p = jnp.concatenate(
        [graph_x, jnp.zeros((n_pad - n, emb), jnp.float32)], axis=0)

    src = _i32(edge_index[0])
    dst = _i32(edge_index[1])
    pad = e_pad - e
    if pad:
        src = jnp.concatenate([src, jnp.zeros((pad,), jnp.int32)])
        dst = jnp.concatenate([dst, jnp.zeros((pad,), jnp.int32)])
        w = jnp.concatenate([edge_weight, jnp.zeros((pad,), jnp.float32)])
    else:
        w = edge_weight
    src_r = src.reshape(NS, chunks, K)
    dst_r = dst.reshape(NS, chunks, K)
    w_r = w.reshape(NS, chunks, K)

    sc_agg_a = _make_sc_agg(n_pad, emb, nblk0, nblk1, compute_deg=True)
    sc_agg_c = _make_sc_agg(n_pad, emb, nblk0, nblk1, compute_deg=False)

    aggp0, degp = sc_agg_a(x_p, src_r, dst_r, w_r)
    degp3 = degp.reshape(NC, n_pad, 1)

    b0r = b0.reshape(1, emb)
    b1r = b1.reshape(1, emb)
    h1 = _tc_layer(x_p, aggp0, degp3, W_root0, W_neigh0, b0r)
    (aggp1,) = sc_agg_c(h1, src_r, dst_r, w_r)
    h2 = _tc_layer(h1, aggp1, degp3, W_root1, W_neigh1, b1r)

    qpt = bq // NW
    ia = _i32(x_nodes[:, 0]).reshape(NW, qpt)
    ib = _i32(x_nodes[:, 1]).reshape(NW, qpt)
    ie = _i32(effect_ids).reshape(NW, qpt)

    sc_pp = _make_sc_pairprod(emb, bq)
    p = sc_pp(h2, effect_table, ia, ib, ie)
    return _tc_decode(p, dec_W, dec_b.reshape(1, 1))


# speedup vs baseline: 1.1433x; 1.0005x over previous
import functools

import jax
import numpy as np
from jax import lax as _lax


def _lane_bcast(v16, lane):
    idx = jnp.full((L, 1), lane, jnp.int32)
    dnums = _lax.GatherDimensionNumbers(
        offset_dims=(), collapsed_slice_dims=(0,), start_index_map=(0,))
    return _lax.gather(v16, idx, dimension_numbers=dnums, slice_sizes=(1,),
                       mode=_lax.GatherScatterMode.PROMISE_IN_BOUNDS)
import jax.numpy as jnp
from jax import lax
from jax.experimental import pallas as pl
from jax.experimental.pallas import tpu as pltpu
from jax.experimental.pallas import tpu_sc as plsc

NC = 2
NS = 16
NW = NC * NS
L = 16
K = 64
NBUF = 4


def _i32(x):
    return x.astype(jnp.int32)


BC = 16


def _make_sc_agg(n_pad, emb, nblk0, nblk1, compute_deg):
    mesh = plsc.VectorSubcoreMesh(core_axis_name="c", subcore_axis_name="s",
                                  num_cores=NC, num_subcores=NS)
    stripe = n_pad // NS

    out_type = [jax.ShapeDtypeStruct((NC, n_pad, emb), jnp.float32)]
    if compute_deg:
        out_type.append(jax.ShapeDtypeStruct((NC, n_pad), jnp.float32))

    scratch = dict(
        src_v=pltpu.VMEM((BC, K), jnp.int32),
        dst_v=pltpu.VMEM((BC, K), jnp.int32),
        w_v=pltpu.VMEM((BC, K), jnp.float32),
        agg_sh=pltpu.VMEM_SHARED((n_pad, emb), jnp.float32),
        sd=pltpu.SemaphoreType.DMA,
        **{f"rows{r}": pltpu.VMEM((K, emb), jnp.float32) for r in range(NBUF)},
        **{f"sg{r}": pltpu.SemaphoreType.DMA for r in range(NBUF)},
        **{f"ss{r}": pltpu.SemaphoreType.DMA for r in range(NBUF)},
    )
    if compute_deg:
        scratch.update(
            zbuf_v=pltpu.VMEM((stripe,), jnp.float32),
            deg_sh=pltpu.VMEM_SHARED((n_pad,), jnp.float32),
        )

    def body(x_hbm, src_hbm, dst_hbm, w_hbm, *outs, src_v, dst_v, w_v,
             agg_sh, sd, zbuf_v=None, deg_sh=None, **bufs):
        if compute_deg:
            agg_out, deg_out = outs
        else:
            (agg_out,) = outs
        c = lax.axis_index("c")
        s = lax.axis_index("s")
        nblk_c = jnp.where(c == 0, nblk0, nblk1)
        bbase = jnp.where(c == 0, 0, nblk0)
        rows = [bufs[f"rows{r}"] for r in range(NBUF)]
        sg = [bufs[f"sg{r}"] for r in range(NBUF)]
        ss = [bufs[f"ss{r}"] for r in range(NBUF)]

        with jax.named_scope("agg_zero"):
            def zrow(e, _):
                for j in range(emb // L):
                    rows[0][e, pl.ds(j * L, L)] = jnp.zeros((L,), jnp.float32)
                return 0
            lax.fori_loop(0, K, zrow, 0)
            for q in range(stripe // K):
                off = s * stripe + q * K
                pltpu.sync_copy(rows[0], agg_sh.at[pl.ds(off, K)])
            if compute_deg:
                for i in range(stripe // L):
                    zbuf_v[pl.ds(i * L, L)] = jnp.zeros((L,), jnp.float32)
                pltpu.sync_copy(zbuf_v, deg_sh.at[pl.ds(s * stripe, stripe)])
            plsc.subcore_barrier()

        def _scale_rows(buf, j):
            def scale(g, _):
                w16 = w_v[j, pl.ds(g * L, L)]
                for l in range(L):
                    e = g * L + l
                    wb = _lane_bcast(w16, l)
                    for jj in range(emb // L):
                        sl = pl.ds(jj * L, L)
                        rows[buf][e, sl] = rows[buf][e, sl] * wb
                return 0
            lax.fori_loop(0, K // L, scale, 0)

        def _drain_scatters():
            for par in range(NBUF):
                pltpu.make_async_copy(
                    rows[par], agg_sh.at[dst_v.at[0]], ss[par]).wait()

        def _drain_deg():
            for _j in range(BC):
                pltpu.make_async_copy(
                    w_v.at[0], deg_sh.at[dst_v.at[0]], sd).wait()

        def block(b, _):
            @pl.when(b > 0)
            def _():
                _drain_scatters()
                if compute_deg:
                    _drain_deg()

            boff = pl.multiple_of((bbase + b) * BC, BC)
            pltpu.sync_copy(src_hbm.at[s, pl.ds(boff, BC)], src_v)
            pltpu.sync_copy(dst_hbm.at[s, pl.ds(boff, BC)], dst_v)
            pltpu.sync_copy(w_hbm.at[s, pl.ds(boff, BC)], w_v)

            gd = [None] * NBUF
            for j0 in range(NBUF - 1):
                gd[j0] = pltpu.async_copy(
                    x_hbm.at[src_v.at[j0]], rows[j0], sg[j0])
            for j in range(BC):
                buf = j % NBUF
                if j + NBUF - 1 < BC:
                    nb = (j + NBUF - 1) % NBUF
                    if j > 0:
                        pltpu.make_async_copy(
                            rows[nb], agg_sh.at[dst_v.at[0]], ss[nb]).wait()
                    gd[nb] = pltpu.async_copy(
                        x_hbm.at[src_v.at[j + NBUF - 1]], rows[nb], sg[nb])
                gd[buf].wait()
                _scale_rows(buf, j)
                pltpu.async_copy(rows[buf], agg_sh.at[dst_v.at[j]], ss[buf],
                                 add=True)
                if compute_deg:
                    pltpu.async_copy(w_v.at[j], deg_sh.at[dst_v.at[j]], sd,
                                     add=True)
            return 0
        with jax.named_scope("agg_main"):
            lax.fori_loop(0, nblk_c, block, 0)
            _drain_scatters()
            if compute_deg:
                _drain_deg()
        with jax.named_scope("agg_bar"):
            plsc.subcore_barrier()

        with jax.named_scope("agg_out"):
            off = s * stripe
            pltpu.sync_copy(agg_sh.at[pl.ds(off, stripe)],
                            agg_out.at[c, pl.ds(off, stripe)])
            if compute_deg:
                pltpu.sync_copy(deg_sh.at[pl.ds(off, stripe)],
                                deg_out.at[c, pl.ds(off, stripe)])

    return pl.kernel(body, out_type=tuple(out_type), mesh=mesh,
                     scratch_types=scratch)


def _tc_layer(x, aggp, degp3, w_root, w_neigh, b, blk=1024):
    n_pad, emb = x.shape
    grid = n_pad // blk

    def body(x_ref, a_ref, d_ref, wr_ref, wn_ref, b_ref, o_ref):
        agg = a_ref[0] + a_ref[1]
        deg = d_ref[0] + d_ref[1]
        inv = 1.0 / jnp.maximum(deg, 1e-12)
        h = (jnp.dot(x_ref[...], wr_ref[...],
                     preferred_element_type=jnp.float32,
                     precision=lax.Precision.HIGHEST)
             + jnp.dot(agg * inv, wn_ref[...],
                       preferred_element_type=jnp.float32,
                       precision=lax.Precision.HIGHEST)
             + b_ref[...])
        o_ref[...] = jnp.maximum(h, 0.0)

    return pl.pallas_call(
        body,
        grid=(grid,),
        in_specs=[
            pl.BlockSpec((blk, emb), lambda i: (i, 0)),
            pl.BlockSpec((NC, blk, emb), lambda i: (0, i, 0)),
            pl.BlockSpec((NC, blk, 1), lambda i: (0, i, 0)),
            pl.BlockSpec((emb, emb), lambda i: (0, 0)),
            pl.BlockSpec((emb, emb), lambda i: (0, 0)),
            pl.BlockSpec((1, emb), lambda i: (0, 0)),
        ],
        out_specs=pl.BlockSpec((blk, emb), lambda i: (i, 0)),
        out_shape=jax.ShapeDtypeStruct((n_pad, emb), jnp.float32),
    )(x, aggp, degp3, w_root, w_neigh, b)


def _make_sc_pairprod(emb, bq):
    mesh = plsc.VectorSubcoreMesh(core_axis_name="c", subcore_axis_name="s",
                                  num_cores=NC, num_subcores=NS)
    qpt = bq // NW

    scratch = dict(
        ia_v=pltpu.VMEM((qpt,), jnp.int32),
        ib_v=pltpu.VMEM((qpt,), jnp.int32),
        ie_v=pltpu.VMEM((qpt,), jnp.int32),
        ha_v=pltpu.VMEM((qpt, emb), jnp.float32),
        hb_v=pltpu.VMEM((qpt, emb), jnp.float32),
        ef_v=pltpu.VMEM((qpt, emb), jnp.float32),
        sem=pltpu.SemaphoreType.DMA,
    )

    def body(h_hbm, eff_hbm, ia_hbm, ib_hbm, ie_hbm, p_hbm,
             *, ia_v, ib_v, ie_v, ha_v, hb_v, ef_v, sem):
        c = lax.axis_index("c")
        s = lax.axis_index("s")
        wid = c * NS + s

        pltpu.sync_copy(ia_hbm.at[wid], ia_v)
        pltpu.sync_copy(ib_hbm.at[wid], ib_v)
        pltpu.sync_copy(ie_hbm.at[wid], ie_v)

        ca = pltpu.async_copy(h_hbm.at[ia_v], ha_v, sem)
        cb = pltpu.async_copy(h_hbm.at[ib_v], hb_v, sem)
        ce = pltpu.async_copy(eff_hbm.at[ie_v], ef_v, sem)
        ca.wait()
        cb.wait()
        ce.wait()

        def qstep(q, _):
            for j in range(emb // L):
                sl = pl.ds(j * L, L)
                a = ha_v[q, sl]
                bb = hb_v[q, sl]
                ee = ef_v[q, sl]
                ha_v[q, sl] = (a + ee) * (bb + ee)
            return 0
        lax.fori_loop(0, qpt, qstep, 0)

        pltpu.sync_copy(ha_v, p_hbm.at[pl.ds(wid * qpt, qpt)])

    return pl.kernel(body,
                     out_type=jax.ShapeDtypeStruct((bq, emb), jnp.float32),
                     mesh=mesh, scratch_types=scratch)


def _tc_decode(p, dec_w, dec_b2):
    bq, emb = p.shape

    def body(p_ref, w_ref, b_ref, o_ref):
        z = jnp.dot(p_ref[...], w_ref[...],
                    preferred_element_type=jnp.float32,
                    precision=lax.Precision.HIGHEST) + b_ref[...]
        o_ref[...] = 1.0 / (1.0 + jnp.exp(-z))

    return pl.pallas_call(
        body,
        out_shape=jax.ShapeDtypeStruct((bq, 1), jnp.float32),
    )(p, dec_w, dec_b2)


def kernel(graph_x, edge_index, edge_weight, x_nodes, effect_ids, effect_table,
           W_root0, W_neigh0, b0, W_root1, W_neigh1, b1, dec_W, dec_b):
    n, emb = graph_x.shape
    e = edge_weight.shape[0]
    bq = x_nodes.shape[0]

    n_pad = ((n + NS * K - 1) // (NS * K)) * (NS * K)
    epw = NS * K * BC
    nblk_t = (e + epw - 1) // epw
    e_pad = nblk_t * epw
    chunks = e_pad // (NS * K)
    nblk0 = max(1, min(nblk_t - 1, round(nblk_t * 0.70)))
    nblk1 = nblk_t - nblk0

    x_p = jnp.concatenate(
        [graph_x, jnp.zeros((n_pad - n, emb), jnp.float32)], axis=0)

    src = _i32(edge_index[0])
    dst = _i32(edge_index[1])
    pad = e_pad - e
    if pad:
        src = jnp.concatenate([src, jnp.zeros((pad,), jnp.int32)])
        dst = jnp.concatenate([dst, jnp.zeros((pad,), jnp.int32)])
        w = jnp.concatenate([edge_weight, jnp.zeros((pad,), jnp.float32)])
    else:
        w = edge_weight
    src_r = src.reshape(NS, chunks, K)
    dst_r = dst.reshape(NS, chunks, K)
    w_r = w.reshape(NS, chunks, K)

    sc_agg_a = _make_sc_agg(n_pad, emb, nblk0, nblk1, compute_deg=True)
    sc_agg_c = _make_sc_agg(n_pad, emb, nblk0, nblk1, compute_deg=False)

    aggp0, degp = sc_agg_a(x_p, src_r, dst_r, w_r)
    degp3 = degp.reshape(NC, n_pad, 1)

    b0r = b0.reshape(1, emb)
    b1r = b1.reshape(1, emb)
    h1 = _tc_layer(x_p, aggp0, degp3, W_root0, W_neigh0, b0r)
    (aggp1,) = sc_agg_c(h1, src_r, dst_r, w_r)
    h2 = _tc_layer(h1, aggp1, degp3, W_root1, W_neigh1, b1r)

    qpt = bq // NW
    ia = _i32(x_nodes[:, 0]).reshape(NW, qpt)
    ib = _i32(x_nodes[:, 1]).reshape(NW, qpt)
    ie = _i32(effect_ids).reshape(NW, qpt)

    sc_pp = _make_sc_pairprod(emb, bq)
    p = sc_pp(h2, effect_table, ia, ib, ie)
    return _tc_decode(p, dec_W, dec_b.reshape(1, 1))
